# Initial kernel scaffold; baseline (speedup 1.0000x reference)
import functools

import jax
import jax.numpy as jnp
from jax import lax
from jax.experimental import pallas as pl
from jax.experimental.pallas import tpu as pltpu
from jax.experimental.pallas import tpu_sc as plsc

N = 10000
E = 320000
D = 128

NC = 2
NS = 16
NW = NC * NS
EPW = E // NW
CH = 80
NCHUNK = EPW // CH
RPS = N // NS


def _sc_segment_sum_body(x_hbm, src_hbm, dst_hbm, zeros_hbm, out_hbm,
                         src_v, dst_v, rows_v, sem, acc):
    c = lax.axis_index("c")
    s = lax.axis_index("s")
    wid = s * NC + c

    pltpu.sync_copy(zeros_hbm.at[pl.ds(s * RPS, RPS)],
                    acc.at[pl.ds(s * RPS, RPS)])
    plsc.subcore_barrier()

    base = wid * EPW

    @pl.loop(0, NCHUNK)
    def _chunk(j):
        off = base + j * CH
        pltpu.sync_copy(src_hbm.at[pl.ds(off, CH)], src_v)
        pltpu.sync_copy(dst_hbm.at[pl.ds(off, CH)], dst_v)
        pltpu.async_copy(x_hbm.at[src_v], rows_v, sem).wait()
        pltpu.sync_copy(rows_v, acc.at[dst_v], add=True)

    plsc.subcore_barrier()
    pltpu.sync_copy(acc.at[pl.ds(s * RPS, RPS)],
                    out_hbm.at[c, pl.ds(s * RPS, RPS)])


_sc_segment_sum = functools.partial(
    pl.kernel,
    out_type=jax.ShapeDtypeStruct((NC, N, D), jnp.float32),
    mesh=plsc.VectorSubcoreMesh(core_axis_name="c", subcore_axis_name="s"),
    scratch_types=[
        pltpu.VMEM((CH,), jnp.int32),
        pltpu.VMEM((CH,), jnp.int32),
        pltpu.VMEM((CH, D), jnp.float32),
        pltpu.SemaphoreType.DMA,
        pltpu.VMEM_SHARED((N, D), jnp.float32),
    ],
)(_sc_segment_sum_body)


def _tc_combine_body(x_ref, p0_ref, p1_ref, w1t_ref, w2t_ref, b2_ref, o_ref):
    f1 = p0_ref[...] + p1_ref[...]
    o_ref[...] = (
        jnp.dot(x_ref[...], w2t_ref[...], preferred_element_type=jnp.float32)
        + b2_ref[...]
        + jnp.dot(f1, w1t_ref[...], preferred_element_type=jnp.float32)
    )


def kernel(x, edge_index, W1, W2, b2):
    src = edge_index[0]
    dst = edge_index[1]
    zeros = jnp.zeros((N, D), dtype=jnp.float32)

    partials = _sc_segment_sum(x, src, dst, zeros)

    out = pl.pallas_call(
        _tc_combine_body,
        out_shape=jax.ShapeDtypeStruct((N, D), jnp.float32),
    )(x, partials[0], partials[1], W1.T, W2.T, b2.reshape(1, D))
    return out


# speedup vs baseline: 5.4184x; 5.4184x over previous
import functools

import jax
import jax.numpy as jnp
from jax import lax
from jax.experimental import pallas as pl
from jax.experimental.pallas import tpu as pltpu
from jax.experimental.pallas import tpu_sc as plsc

N = 10000
E = 320000
D = 128

NC = 2
NS = 16
NW = NC * NS
EPW = E // NW
CH = 80
NCHUNK = EPW // CH
N_PAD = 10240
RPS = N_PAD // NS


def _sc_segment_sum_body(x_hbm, src_hbm, dst_hbm, zeros_hbm, out_hbm,
                         src_v, dst_v, rows_v, sem, acc):
    c = lax.axis_index("c")
    s = lax.axis_index("s")
    wid = s * NC + c

    pltpu.sync_copy(zeros_hbm, acc.at[pl.ds(s * RPS, RPS)])
    plsc.subcore_barrier()

    base = wid * EPW

    @pl.loop(0, NCHUNK)
    def _chunk(j):
        off = base + j * CH
        pltpu.sync_copy(src_hbm.at[pl.ds(off, CH)], src_v)
        pltpu.sync_copy(dst_hbm.at[pl.ds(off, CH)], dst_v)
        pltpu.async_copy(x_hbm.at[src_v], rows_v, sem).wait()
        pltpu.sync_copy(rows_v, acc.at[dst_v], add=True)

    plsc.subcore_barrier()
    pltpu.sync_copy(acc.at[pl.ds(s * RPS, RPS)],
                    out_hbm.at[c, pl.ds(s * RPS, RPS)])


@functools.lru_cache(maxsize=None)
def _sc_segment_sum():
    return pl.kernel(
        _sc_segment_sum_body,
        out_type=jax.ShapeDtypeStruct((NC, N_PAD, D), jnp.float32),
        mesh=plsc.VectorSubcoreMesh(core_axis_name="c", subcore_axis_name="s",
                                    num_cores=NC, num_subcores=NS),
        scratch_types=[
            pltpu.VMEM((CH,), jnp.int32),
            pltpu.VMEM((CH,), jnp.int32),
            pltpu.VMEM((CH, D), jnp.float32),
            pltpu.SemaphoreType.DMA,
            pltpu.VMEM_SHARED((N_PAD, D), jnp.float32),
        ],
    )


def _tc_combine_body(x_ref, p0_ref, p1_ref, w1t_ref, w2t_ref, b2_ref, o_ref):
    f1 = p0_ref[...] + p1_ref[...]
    o_ref[...] = (
        jnp.dot(x_ref[...], w2t_ref[...], preferred_element_type=jnp.float32)
        + b2_ref[...]
        + jnp.dot(f1, w1t_ref[...], preferred_element_type=jnp.float32)
    )


def kernel(x, edge_index, W1, W2, b2):
    src = edge_index[0]
    dst = edge_index[1]
    zeros = jnp.zeros((RPS, D), dtype=jnp.float32)

    partials = _sc_segment_sum()(x, src, dst, zeros)

    out = pl.pallas_call(
        _tc_combine_body,
        out_shape=jax.ShapeDtypeStruct((N, D), jnp.float32),
    )(x, partials[0, :N], partials[1, :N], W1.T, W2.T, b2.reshape(1, D))
    return out
